# Initial kernel scaffold; baseline (speedup 1.0000x reference)
import jax
import jax.numpy as jnp
from jax.experimental import pallas as pl


def kernel(x, hyperedge_index, hyperedge_type, A1, C1_w, C1_b, A2, C2_w, C2_b, L1_w, L1_b, L2_w, L2_b):
    raise NotImplementedError("write your pallas kernel here")


# speedup vs baseline: 11.2411x; 11.2411x over previous
import functools

import jax
import jax.numpy as jnp
from jax import lax
from jax.experimental import pallas as pl
from jax.experimental.pallas import tpu as pltpu
from jax.experimental.pallas import tpu_sc as plsc

N = 10000
NP = 10240
D = 128
T = 4
NTP = T * NP
ACC_R = NTP + 128
E = 320000
NTILE = 16
EPT = E // NTILE
CH = 1024
NFULL = EPT // CH
TAIL = EPT - NFULL * CH
QROWS = NTP // NTILE
ZR = ACC_R // NTILE
BN = 512
NB = NP // BN


def _fill_oidx(oidx, dstb, etb, ng, nslots):
    for g in range(ng):
        j, c = g // 8, (g % 8) * 16
        oidx[j, pl.ds(c, 16)] = (etb[pl.ds(g * 16, 16)] * NP
                                 + dstb[pl.ds(g * 16, 16)])
    for g in range(ng, nslots):
        j, c = g // 8, (g % 8) * 16
        oidx[j, pl.ds(c, 16)] = jnp.full((16,), NTP, jnp.int32)


def _sc_accum_body(xs, srcv, dstv, etv, zref,
                   s_out, acc, srcb, dstb, etb, gidx, oidx, rows, sem):
    cid = lax.axis_index("c")
    sid = lax.axis_index("s")
    ebase = sid * EPT

    def do_chunk(eoff, nv, qbase):
        eoff = pl.multiple_of(eoff, 8)
        pltpu.sync_copy(srcv.at[pl.ds(eoff, nv)], srcb.at[pl.ds(0, nv)])
        pltpu.sync_copy(dstv.at[pl.ds(eoff, nv)], dstb.at[pl.ds(0, nv)])
        pltpu.sync_copy(etv.at[pl.ds(eoff, nv)], etb.at[pl.ds(0, nv)])
        ng = nv // 16
        for g in range(ng):
            j, c = g // 8, (g % 8) * 16
            gidx[j, pl.ds(c, 16)] = srcb[pl.ds(g * 16, 16)] + qbase
        for g in range(ng, CH // 16):
            j, c = g // 8, (g % 8) * 16
            gidx[j, pl.ds(c, 16)] = jnp.zeros((16,), jnp.int32)
        _fill_oidx(oidx, dstb, etb, ng, CH // 16)
        cps = [pltpu.async_copy(xs.at[gidx.at[j]],
                                rows.at[pl.ds(j * 128, 128)], sem)
               for j in range(8)]
        for cp in cps:
            cp.wait()
        for j in range(8):
            pltpu.sync_copy(rows.at[pl.ds(j * 128, 128)],
                            acc.at[oidx.at[j]], add=True)

    for p in range(2):
        q = cid * 2 + p
        qbase = q * NP
        zoff = pl.multiple_of(sid * ZR, 8)
        pltpu.sync_copy(zref, acc.at[pl.ds(zoff, ZR)])
        plsc.subcore_barrier()

        def body(k, carry):
            do_chunk(ebase + k * CH, CH, qbase)
            return carry
        lax.fori_loop(0, NFULL, body, 0)
        do_chunk(ebase + NFULL * CH, TAIL, qbase)
        plsc.subcore_barrier()

        doff = pl.multiple_of(sid * QROWS, 8)
        pltpu.sync_copy(acc.at[pl.ds(doff, QROWS)],
                        s_out.at[q, pl.ds(doff, QROWS)])
        plsc.subcore_barrier()


EPW = E // 32
CFULL = EPW // CH
CTAIL = EPW - CFULL * CH


def _sc_counts_body(dstv, etv, z16, ones_h,
                    cnt_out, cacc, dstb, etb, oidx, ones):
    cid = lax.axis_index("c")
    sid = lax.axis_index("s")
    wid = cid * NTILE + sid
    ebase = wid * EPW

    pltpu.sync_copy(ones_h, ones)
    zoff = pl.multiple_of(sid * ZR, 8)
    pltpu.sync_copy(z16, cacc.at[pl.ds(zoff, ZR)])
    plsc.subcore_barrier()

    def do_chunk(eoff, nv):
        eoff = pl.multiple_of(eoff, 8)
        pltpu.sync_copy(dstv.at[pl.ds(eoff, nv)], dstb.at[pl.ds(0, nv)])
        pltpu.sync_copy(etv.at[pl.ds(eoff, nv)], etb.at[pl.ds(0, nv)])
        _fill_oidx(oidx, dstb, etb, nv // 16, CH // 16)
        for j in range(8):
            pltpu.sync_copy(ones, cacc.at[oidx.at[j]], add=True)

    def body(k, carry):
        do_chunk(ebase + k * CH, CH)
        return carry
    lax.fori_loop(0, CFULL, body, 0)
    do_chunk(ebase + CFULL * CH, CTAIL)
    plsc.subcore_barrier()

    doff = pl.multiple_of(sid * QROWS, 8)
    pltpu.sync_copy(cacc.at[pl.ds(doff, QROWS)],
                    cnt_out.at[cid, pl.ds(doff, QROWS)])


def _make_mesh():
    return plsc.VectorSubcoreMesh(core_axis_name="c", subcore_axis_name="s")


_sc_accum = functools.partial(
    pl.kernel,
    out_type=jax.ShapeDtypeStruct((T, NTP, 32), jnp.float32),
    mesh=_make_mesh(),
    scratch_types=[
        pltpu.VMEM_SHARED((ACC_R, 32), jnp.float32),
        pltpu.VMEM((CH,), jnp.int32),
        pltpu.VMEM((CH,), jnp.int32),
        pltpu.VMEM((CH,), jnp.int32),
        pltpu.VMEM((8, 128), jnp.int32),
        pltpu.VMEM((8, 128), jnp.int32),
        pltpu.VMEM((CH, 32), jnp.float32),
        pltpu.SemaphoreType.DMA,
    ],
    compiler_params=pltpu.CompilerParams(use_tc_tiling_on_sc=False),
)(_sc_accum_body)


_sc_counts = functools.partial(
    pl.kernel,
    out_type=jax.ShapeDtypeStruct((2, NTP, 16), jnp.float32),
    mesh=_make_mesh(),
    scratch_types=[
        pltpu.VMEM_SHARED((ACC_R, 16), jnp.float32),
        pltpu.VMEM((CH,), jnp.int32),
        pltpu.VMEM((CH,), jnp.int32),
        pltpu.VMEM((8, 128), jnp.int32),
        pltpu.VMEM((128, 16), jnp.float32),
    ],
    compiler_params=pltpu.CompilerParams(use_tc_tiling_on_sc=False),
)(_sc_counts_body)


def _tc1_body(x_ref, s_ref, inv_ref, a_ref, cw_ref, b_ref, out_ref, acc_ref):
    t = pl.program_id(1)
    m = jnp.concatenate([s_ref[q] for q in range(4)], axis=1)
    m = m * inv_ref[0]
    part = jnp.dot(m, a_ref[0], preferred_element_type=jnp.float32)

    @pl.when(t == 0)
    def _():
        acc_ref[...] = (jnp.dot(x_ref[...], cw_ref[...],
                                preferred_element_type=jnp.float32)
                        + b_ref[...])
    acc_ref[...] += part

    @pl.when(t == 3)
    def _():
        h = jnp.maximum(acc_ref[...], 0.0)
        for qq in range(4):
            out_ref[qq] = h[:, qq * 32:(qq + 1) * 32]


def _tc2_body(hq_ref, s_ref, inv_ref, a_ref, cw_ref, b_ref,
              l1w_ref, l1b_ref, l2w_ref, l2b_ref, out_ref, acc_ref):
    t = pl.program_id(1)
    m = jnp.concatenate([s_ref[q] for q in range(4)], axis=1)
    m = m * inv_ref[0]
    part = jnp.dot(m, a_ref[0], preferred_element_type=jnp.float32)

    @pl.when(t == 0)
    def _():
        xin = jnp.concatenate([hq_ref[q] for q in range(4)], axis=1)
        acc_ref[...] = (jnp.dot(xin, cw_ref[...],
                                preferred_element_type=jnp.float32)
                        + b_ref[...])
    acc_ref[...] += part

    @pl.when(t == 3)
    def _():
        h2 = jnp.maximum(acc_ref[...], 0.0)
        h3 = jnp.maximum(jnp.dot(h2, l1w_ref[...],
                                 preferred_element_type=jnp.float32)
                         + l1b_ref[...], 0.0)
        z = (jnp.dot(h3, l2w_ref[...], preferred_element_type=jnp.float32)
             + l2b_ref[...])
        out_ref[...] = jax.nn.sigmoid(z)


_S_SPEC = pl.BlockSpec((4, BN, 32), lambda i, t: (0, t * NB + i, 0))
_HQ_SPEC = pl.BlockSpec((4, BN, 32), lambda i, t: (0, i, 0))
_INV_SPEC = pl.BlockSpec((1, BN, D), lambda i, t: (t, i, 0))
_A_SPEC = pl.BlockSpec((1, D, D), lambda i, t: (t, 0, 0))
_W_SPEC = pl.BlockSpec((D, D), lambda i, t: (0, 0))
_B_SPEC = pl.BlockSpec((1, D), lambda i, t: (0, 0))

_tc1 = pl.pallas_call(
    _tc1_body,
    grid=(NB, 4),
    in_specs=[pl.BlockSpec((BN, D), lambda i, t: (i, 0)),
              _S_SPEC, _INV_SPEC, _A_SPEC, _W_SPEC, _B_SPEC],
    out_specs=_HQ_SPEC,
    out_shape=jax.ShapeDtypeStruct((4, NP, 32), jnp.float32),
    scratch_shapes=[pltpu.VMEM((BN, D), jnp.float32)],
)

_tc2 = pl.pallas_call(
    _tc2_body,
    grid=(NB, 4),
    in_specs=[_HQ_SPEC, _S_SPEC, _INV_SPEC, _A_SPEC, _W_SPEC, _B_SPEC,
              _W_SPEC, _B_SPEC, _W_SPEC, _B_SPEC],
    out_specs=pl.BlockSpec((BN, D), lambda i, t: (i, 0)),
    out_shape=jax.ShapeDtypeStruct((NP, D), jnp.float32),
    scratch_shapes=[pltpu.VMEM((BN, D), jnp.float32)],
)


def kernel(x, hyperedge_index, hyperedge_type,
           A1, C1_w, C1_b, A2, C2_w, C2_b, L1_w, L1_b, L2_w, L2_b):
    src = hyperedge_index[0]
    dst = hyperedge_index[1]
    et = hyperedge_type.astype(jnp.int32)

    x_pad = jnp.pad(x, ((0, NP - N), (0, 0)))
    xs1 = x_pad.reshape(NP, 4, 32).transpose(1, 0, 2).reshape(T * NP, 32)

    zref = jnp.zeros((ZR, 32), jnp.float32)
    z16 = jnp.zeros((ZR, 16), jnp.float32)
    ones_h = jnp.ones((128, 16), jnp.float32)

    cnt2 = _sc_counts(dst, et, z16, ones_h)
    cnt = cnt2[0, :, 0] + cnt2[1, :, 0]
    inv = 1.0 / jnp.maximum(cnt, 1.0)
    inv_rep = jnp.broadcast_to(inv.reshape(T, NP, 1), (T, NP, D))

    S1 = _sc_accum(xs1, src, dst, et, zref)

    h1q = _tc1(x_pad, S1, inv_rep, A1, C1_w.T, C1_b.reshape(1, D))

    xs2 = h1q.reshape(T * NP, 32)
    S2 = _sc_accum(xs2, src, dst, et, zref)

    l2w = jnp.pad(L2_w.T, ((0, 0), (0, D - 1)))
    l2b = jnp.broadcast_to(L2_b.reshape(1, 1), (1, D))
    out_pad = _tc2(h1q, S2, inv_rep, A2, C2_w.T, C2_b.reshape(1, D),
                   L1_w.T, L1_b.reshape(1, D), l2w, l2b)
    return out_pad[:N, :1]


# speedup vs baseline: 19.7794x; 1.7596x over previous
import functools

import jax
import jax.numpy as jnp
from jax import lax
from jax.experimental import pallas as pl
from jax.experimental.pallas import tpu as pltpu
from jax.experimental.pallas import tpu_sc as plsc

N = 10000
NP = 10240
D = 128
T = 4
NTP = T * NP
ACC_R = NTP + 128
E = 320000
NTILE = 16
EPT = E // NTILE
CH = 512
QROWS = NTP // NTILE
ZR = ACC_R // NTILE
BN = 512
NB = NP // BN


def _fill_oidx(oidx, dstb, etb, ng, nslots):
    for g in range(ng):
        j, c = g // 8, (g % 8) * 16
        oidx[j, pl.ds(c, 16)] = (etb[pl.ds(g * 16, 16)] * NP
                                 + dstb[pl.ds(g * 16, 16)])
    for g in range(ng, nslots):
        j, c = g // 8, (g % 8) * 16
        oidx[j, pl.ds(c, 16)] = jnp.full((16,), NTP, jnp.int32)


NST = CH // 128
NCHUNK = EPT // CH
TAILC = EPT - NCHUNK * CH
PAIRS = (NCHUNK - 3) // 2


def _sc_accum_body(xs, srcv, dstv, etv, zref,
                   s_out, acc,
                   srcb0, dstb0, etb0, srcb1, dstb1, etb1,
                   gidx0, oidx0, gidx1, oidx1, rows0, rows1,
                   sem_i0, sem_i1, sem_g, sem_s0, sem_s1):
    cid = lax.axis_index("c")
    sid = lax.axis_index("s")
    ebase = sid * EPT
    bufs = ((srcb0, dstb0, etb0, gidx0, oidx0, rows0, sem_i0, sem_s0),
            (srcb1, dstb1, etb1, gidx1, oidx1, rows1, sem_i1, sem_s1))

    def issue_idx(eoff, nv, b):
        srcb, dstb, etb, _, _, _, sem_i, _ = bufs[b]
        eoff = pl.multiple_of(eoff, 8)
        pltpu.async_copy(srcv.at[pl.ds(eoff, nv)], srcb.at[pl.ds(0, nv)],
                         sem_i)
        pltpu.async_copy(dstv.at[pl.ds(eoff, nv)], dstb.at[pl.ds(0, nv)],
                         sem_i)
        pltpu.async_copy(etv.at[pl.ds(eoff, nv)], etb.at[pl.ds(0, nv)],
                         sem_i)

    def drain_scatter(b, nstream):
        _, _, _, _, _, rows, _, sem_s = bufs[b]
        for j in range(nstream):
            pltpu.make_async_copy(zref.at[pl.ds(0, 128)],
                                  rows.at[pl.ds(j * 128, 128)],
                                  sem_s).wait()

    def process(nv, b, qbase, drain, nxt):
        srcb, dstb, etb, gidx, oidx, rows, sem_i, sem_s = bufs[b]
        for ref in (srcb, dstb, etb):
            pltpu.make_async_copy(srcv.at[pl.ds(0, nv)],
                                  ref.at[pl.ds(0, nv)], sem_i).wait()
        ng = nv // 16
        for g in range(ng):
            j, c = g // 8, (g % 8) * 16
            gidx[j, pl.ds(c, 16)] = srcb[pl.ds(g * 16, 16)] + qbase
        for g in range(ng, CH // 16):
            j, c = g // 8, (g % 8) * 16
            gidx[j, pl.ds(c, 16)] = jnp.zeros((16,), jnp.int32)
        _fill_oidx(oidx, dstb, etb, ng, CH // 16)
        if nxt is not None:
            issue_idx(*nxt)
        nstream = max(1, nv // 128)
        if drain is True:
            drain_scatter(b, NST)
        elif drain is not None:
            pl.when(drain)(lambda: drain_scatter(b, NST))
        cps = [pltpu.async_copy(xs.at[gidx.at[j]],
                                rows.at[pl.ds(j * 128, 128)], sem_g)
               for j in range(nstream)]
        for cp in cps:
            cp.wait()
        for j in range(nstream):
            pltpu.async_copy(rows.at[pl.ds(j * 128, 128)],
                             acc.at[oidx.at[j]], sem_s, add=True)

    for p in range(2):
        qbase = (cid * 2 + p) * NP
        zoff = pl.multiple_of(sid * ZR, 8)
        pltpu.sync_copy(zref, acc.at[pl.ds(zoff, ZR)])
        plsc.subcore_barrier()

        issue_idx(ebase, CH, 0)
        issue_idx(ebase + CH, CH, 1)

        def body(i, carry):
            k0 = 2 * i
            process(CH, 0, qbase, i > 0, (ebase + (k0 + 2) * CH, CH, 0))
            process(CH, 1, qbase, i > 0, (ebase + (k0 + 3) * CH, CH, 1))
            return carry
        lax.fori_loop(0, PAIRS, body, 0)
        k = 2 * PAIRS
        process(CH, 0, qbase, True, (ebase + (k + 2) * CH, CH, 0))
        process(CH, 1, qbase, True, (ebase + (k + 3) * CH, TAILC, 1))
        process(CH, 0, qbase, True, None)
        process(TAILC, 1, qbase, True, None)
        drain_scatter(0, NST)
        drain_scatter(1, 1)
        plsc.subcore_barrier()

        doff = pl.multiple_of(sid * QROWS, 8)
        pltpu.sync_copy(acc.at[pl.ds(doff, QROWS)],
                        s_out.at[(cid * 2 + p), pl.ds(doff, QROWS)])
        plsc.subcore_barrier()


EPW = E // 32
CFULL = EPW // CH
CTAIL = EPW - CFULL * CH


def _sc_counts_body(dstv, etv, z16, ones_h,
                    cnt_out, cacc, dstb, etb, oidx, ones):
    cid = lax.axis_index("c")
    sid = lax.axis_index("s")
    wid = cid * NTILE + sid
    ebase = wid * EPW

    pltpu.sync_copy(ones_h, ones)
    zoff = pl.multiple_of(sid * ZR, 8)
    pltpu.sync_copy(z16, cacc.at[pl.ds(zoff, ZR)])
    plsc.subcore_barrier()

    def do_chunk(eoff, nv):
        eoff = pl.multiple_of(eoff, 8)
        pltpu.sync_copy(dstv.at[pl.ds(eoff, nv)], dstb.at[pl.ds(0, nv)])
        pltpu.sync_copy(etv.at[pl.ds(eoff, nv)], etb.at[pl.ds(0, nv)])
        _fill_oidx(oidx, dstb, etb, nv // 16, CH // 16)
        for j in range(NST):
            pltpu.sync_copy(ones, cacc.at[oidx.at[j]], add=True)

    def body(k, carry):
        do_chunk(ebase + k * CH, CH)
        return carry
    lax.fori_loop(0, CFULL, body, 0)
    do_chunk(ebase + CFULL * CH, CTAIL)
    plsc.subcore_barrier()

    doff = pl.multiple_of(sid * QROWS, 8)
    pltpu.sync_copy(cacc.at[pl.ds(doff, QROWS)],
                    cnt_out.at[cid, pl.ds(doff, QROWS)])


def _make_mesh():
    return plsc.VectorSubcoreMesh(core_axis_name="c", subcore_axis_name="s")


_sc_accum = functools.partial(
    pl.kernel,
    out_type=jax.ShapeDtypeStruct((T, NTP, 32), jnp.float32),
    mesh=_make_mesh(),
    scratch_types=(
        [pltpu.VMEM_SHARED((ACC_R, 32), jnp.float32)]
        + [pltpu.VMEM((CH,), jnp.int32) for _ in range(6)]
        + [pltpu.VMEM((NST, 128), jnp.int32) for _ in range(4)]
        + [pltpu.VMEM((CH, 32), jnp.float32) for _ in range(2)]
        + [pltpu.SemaphoreType.DMA for _ in range(5)]
    ),
    compiler_params=pltpu.CompilerParams(use_tc_tiling_on_sc=False),
)(_sc_accum_body)


_sc_counts = functools.partial(
    pl.kernel,
    out_type=jax.ShapeDtypeStruct((2, NTP, 16), jnp.float32),
    mesh=_make_mesh(),
    scratch_types=[
        pltpu.VMEM_SHARED((ACC_R, 16), jnp.float32),
        pltpu.VMEM((CH,), jnp.int32),
        pltpu.VMEM((CH,), jnp.int32),
        pltpu.VMEM((NST, 128), jnp.int32),
        pltpu.VMEM((128, 16), jnp.float32),
    ],
    compiler_params=pltpu.CompilerParams(use_tc_tiling_on_sc=False),
)(_sc_counts_body)


def _tc1_body(x_ref, s_ref, inv_ref, a_ref, cw_ref, b_ref, out_ref, acc_ref):
    t = pl.program_id(1)
    m = jnp.concatenate([s_ref[q] for q in range(4)], axis=1)
    m = m * inv_ref[0]
    part = jnp.dot(m, a_ref[0], preferred_element_type=jnp.float32)

    @pl.when(t == 0)
    def _():
        acc_ref[...] = (jnp.dot(x_ref[...], cw_ref[...],
                                preferred_element_type=jnp.float32)
                        + b_ref[...])
    acc_ref[...] += part

    @pl.when(t == 3)
    def _():
        h = jnp.maximum(acc_ref[...], 0.0)
        for qq in range(4):
            out_ref[qq] = h[:, qq * 32:(qq + 1) * 32]


def _tc2_body(hq_ref, s_ref, inv_ref, a_ref, cw_ref, b_ref,
              l1w_ref, l1b_ref, l2w_ref, l2b_ref, out_ref, acc_ref):
    t = pl.program_id(1)
    m = jnp.concatenate([s_ref[q] for q in range(4)], axis=1)
    m = m * inv_ref[0]
    part = jnp.dot(m, a_ref[0], preferred_element_type=jnp.float32)

    @pl.when(t == 0)
    def _():
        xin = jnp.concatenate([hq_ref[q] for q in range(4)], axis=1)
        acc_ref[...] = (jnp.dot(xin, cw_ref[...],
                                preferred_element_type=jnp.float32)
                        + b_ref[...])
    acc_ref[...] += part

    @pl.when(t == 3)
    def _():
        h2 = jnp.maximum(acc_ref[...], 0.0)
        h3 = jnp.maximum(jnp.dot(h2, l1w_ref[...],
                                 preferred_element_type=jnp.float32)
                         + l1b_ref[...], 0.0)
        z = (jnp.dot(h3, l2w_ref[...], preferred_element_type=jnp.float32)
             + l2b_ref[...])
        out_ref[...] = jax.nn.sigmoid(z)


_S_SPEC = pl.BlockSpec((4, BN, 32), lambda i, t: (0, t * NB + i, 0))
_HQ_SPEC = pl.BlockSpec((4, BN, 32), lambda i, t: (0, i, 0))
_INV_SPEC = pl.BlockSpec((1, BN, D), lambda i, t: (t, i, 0))
_A_SPEC = pl.BlockSpec((1, D, D), lambda i, t: (t, 0, 0))
_W_SPEC = pl.BlockSpec((D, D), lambda i, t: (0, 0))
_B_SPEC = pl.BlockSpec((1, D), lambda i, t: (0, 0))

_tc1 = pl.pallas_call(
    _tc1_body,
    grid=(NB, 4),
    in_specs=[pl.BlockSpec((BN, D), lambda i, t: (i, 0)),
              _S_SPEC, _INV_SPEC, _A_SPEC, _W_SPEC, _B_SPEC],
    out_specs=_HQ_SPEC,
    out_shape=jax.ShapeDtypeStruct((4, NP, 32), jnp.float32),
    scratch_shapes=[pltpu.VMEM((BN, D), jnp.float32)],
)

_tc2 = pl.pallas_call(
    _tc2_body,
    grid=(NB, 4),
    in_specs=[_HQ_SPEC, _S_SPEC, _INV_SPEC, _A_SPEC, _W_SPEC, _B_SPEC,
              _W_SPEC, _B_SPEC, _W_SPEC, _B_SPEC],
    out_specs=pl.BlockSpec((BN, D), lambda i, t: (i, 0)),
    out_shape=jax.ShapeDtypeStruct((NP, D), jnp.float32),
    scratch_shapes=[pltpu.VMEM((BN, D), jnp.float32)],
)


def kernel(x, hyperedge_index, hyperedge_type,
           A1, C1_w, C1_b, A2, C2_w, C2_b, L1_w, L1_b, L2_w, L2_b):
    src = hyperedge_index[0]
    dst = hyperedge_index[1]
    et = hyperedge_type.astype(jnp.int32)

    x_pad = jnp.pad(x, ((0, NP - N), (0, 0)))
    xs1 = x_pad.reshape(NP, 4, 32).transpose(1, 0, 2).reshape(T * NP, 32)

    zref = jnp.zeros((ZR, 32), jnp.float32)
    z16 = jnp.zeros((ZR, 16), jnp.float32)
    ones_h = jnp.ones((128, 16), jnp.float32)

    cnt2 = _sc_counts(dst, et, z16, ones_h)
    cnt = cnt2[0, :, 0] + cnt2[1, :, 0]
    inv = 1.0 / jnp.maximum(cnt, 1.0)
    inv_rep = jnp.broadcast_to(inv.reshape(T, NP, 1), (T, NP, D))

    S1 = _sc_accum(xs1, src, dst, et, zref)

    h1q = _tc1(x_pad, S1, inv_rep, A1, C1_w.T, C1_b.reshape(1, D))

    xs2 = h1q.reshape(T * NP, 32)
    S2 = _sc_accum(xs2, src, dst, et, zref)

    l2w = jnp.pad(L2_w.T, ((0, 0), (0, D - 1)))
    l2b = jnp.broadcast_to(L2_b.reshape(1, 1), (1, D))
    out_pad = _tc2(h1q, S2, inv_rep, A2, C2_w.T, C2_b.reshape(1, D),
                   L1_w.T, L1_b.reshape(1, D), l2w, l2b)
    return out_pad[:N, :1]
